# baseline (speedup 1.0000x reference)
import functools

import jax
import jax.numpy as jnp
from jax import lax
from jax.experimental import pallas as pl
from jax.experimental.pallas import tpu as pltpu, tpu_sc as plsc

N = 10000
E = 320000
C = 128
D = 2 * C
ALPHA = 0.5
BETA = 0.5

NC, NS, L = 2, 16, 16
NW = NC * NS

NP = 10240
EP = 327680
BB = 80
NB0 = 180
NB1 = 76
ET0 = NB0 * BB
ET1 = NB1 * BB
EPAD = EP + (ET0 - ET1)
KC = 125
KR = E // NW // KC

_mesh = plsc.VectorSubcoreMesh(
    core_axis_name="c", subcore_axis_name="s", num_cores=NC, num_subcores=NS)


@functools.partial(
    pl.kernel,
    out_type=jax.ShapeDtypeStruct((NC, 2, NP), jnp.float32),
    mesh=_mesh,
    compiler_params=pltpu.CompilerParams(needs_layout_passes=False),
    scratch_types=[
        pltpu.VMEM((KR, KC), jnp.int32),
        pltpu.VMEM((KR, KC), jnp.int32),
        pltpu.VMEM((128,), jnp.float32),
        pltpu.VMEM_SHARED((NP,), jnp.float32),
        pltpu.VMEM_SHARED((NP,), jnp.float32),
        pltpu.SemaphoreType.DMA,
        pltpu.SemaphoreType.DMA,
    ],
)
def _bincount_kernel(src_hbm, dst_hbm, zeros_hbm, cnt_hbm,
                     idx_s, idx_d, ones_v, cnt_s_sh, cnt_d_sh, sem_s, sem_d):
    cid = lax.axis_index("c")
    sid = lax.axis_index("s")
    wid = sid * NC + cid

    @pl.when(sid == 0)
    def _zero():
        pltpu.sync_copy(zeros_hbm, cnt_s_sh)
        pltpu.sync_copy(zeros_hbm, cnt_d_sh)

    for k in range(128 // L):
        ones_v[pl.ds(k * L, L)] = jnp.ones((L,), jnp.float32)

    pltpu.sync_copy(src_hbm.at[pl.ds(wid * KR, KR)], idx_s)
    pltpu.sync_copy(dst_hbm.at[pl.ds(wid * KR, KR)], idx_d)
    plsc.subcore_barrier()

    ones_c = ones_v.at[pl.ds(0, KC)]

    def _issue(j, carry):
        pltpu.async_copy(ones_c, cnt_s_sh.at[idx_s.at[j]], sem_s, add=True)
        pltpu.async_copy(ones_c, cnt_d_sh.at[idx_d.at[j]], sem_d, add=True)
        return carry
    lax.fori_loop(0, KR, _issue, 0)

    def _drain(j, carry):
        pltpu.make_async_copy(ones_c, cnt_s_sh.at[idx_s.at[0]], sem_s).wait()
        pltpu.make_async_copy(ones_c, cnt_d_sh.at[idx_d.at[0]], sem_d).wait()
        return carry
    lax.fori_loop(0, KR, _drain, 0)

    plsc.subcore_barrier()

    @pl.when(sid == 0)
    def _writeback():
        pltpu.sync_copy(cnt_s_sh, cnt_hbm.at[cid, 0])
        pltpu.sync_copy(cnt_d_sh, cnt_hbm.at[cid, 1])


def _scalars_body(cnts_ref, zol_ref, zil_ref, su_ref, sv_ref):
    cs = cnts_ref[0, 0] + cnts_ref[1, 0]
    cd = cnts_ref[0, 1] + cnts_ref[1, 1]
    occ = cs + cd
    denom = jnp.sum((occ > 0).astype(jnp.float32)) - 1.0
    dego = cs / denom
    dego = dego / jnp.maximum(jnp.sqrt(jnp.sum(dego * dego)), 1e-12)
    degi = cd / denom
    degi = degi / jnp.maximum(jnp.sqrt(jnp.sum(degi * degi)), 1e-12)
    su_ref[...] = BETA * (dego + zol_ref[...])
    sv_ref[...] = BETA * (degi + zil_ref[...])


def _scalars_call(cnts, zol, zil):
    return pl.pallas_call(
        _scalars_body,
        out_shape=(jax.ShapeDtypeStruct((NP // 128, 128), jnp.float32),
                   jax.ShapeDtypeStruct((NP // 128, 128), jnp.float32)),
    )(cnts, zol, zil)


_RB = 1000


def _pack_words(lo_f32, hi_f32):
    lo = lax.bitcast_convert_type(lo_f32.astype(jnp.bfloat16),
                                  jnp.uint16).astype(jnp.uint32)
    hi = lax.bitcast_convert_type(hi_f32.astype(jnp.bfloat16),
                                  jnp.uint16).astype(jnp.uint32)
    return lo | (hi << 16)


def _tables_body(zo_ref, zi_ref, zs_ref, wi_ref, wo_ref, bi_ref, bo_ref,
                 u_ref, v_ref, u2_ref, v2_ref):
    zs = zs_ref[...]
    dn = (((1,), (1,)), ((), ()))
    a_in = lax.dot_general(zs, wi_ref[...], dn,
                           preferred_element_type=jnp.float32) + bi_ref[...]
    a_out = lax.dot_general(zs, wo_ref[...], dn,
                            preferred_element_type=jnp.float32) + bo_ref[...]
    uw = _pack_words(ALPHA * zo_ref[...], ALPHA * a_in)
    vw = _pack_words(a_out, zi_ref[...])
    u_ref[...] = uw
    v_ref[...] = vw
    u2_ref[...] = uw
    v2_ref[...] = vw


def _tables_call(zo, zi, zs, wi, wo, bi, bo):
    nsteps = N // _RB
    row_spec = pl.BlockSpec((_RB, C), lambda i: (i, 0))
    fix2 = pl.BlockSpec((C, C), lambda i: (0, 0))
    fixb = pl.BlockSpec((1, C), lambda i: (0, 0))
    return pl.pallas_call(
        _tables_body,
        grid=(nsteps,),
        in_specs=[row_spec, row_spec, row_spec, fix2, fix2, fixb, fixb],
        out_specs=(pl.BlockSpec((_RB, C), lambda i: (i, 0)),
                   pl.BlockSpec((_RB, C), lambda i: (i, 0)),
                   pl.BlockSpec((_RB, C), lambda i: (i, 0)),
                   pl.BlockSpec((_RB, C), lambda i: (i, 0))),
        out_shape=(jax.ShapeDtypeStruct((N, C), jnp.uint32),
                   jax.ShapeDtypeStruct((N, C), jnp.uint32),
                   jax.ShapeDtypeStruct((N, C), jnp.uint32),
                   jax.ShapeDtypeStruct((N, C), jnp.uint32)),
    )(zo, zi, zs, wi, wo, bi, bo)


@functools.partial(
    pl.kernel,
    out_type=jax.ShapeDtypeStruct((EP,), jnp.float32),
    mesh=_mesh,
    compiler_params=pltpu.CompilerParams(needs_layout_passes=False),
    scratch_types=[
        pltpu.VMEM((ET0,), jnp.int32),
        pltpu.VMEM((ET0,), jnp.int32),
        pltpu.VMEM((NP,), jnp.float32),
        pltpu.VMEM((NP,), jnp.float32),
        pltpu.VMEM((2, BB, C), jnp.uint32),
        pltpu.VMEM((2, BB, C), jnp.uint32),
        pltpu.VMEM((BB * L,), jnp.float32),
        pltpu.VMEM((2, BB), jnp.float32),
        pltpu.SemaphoreType.DMA((2,)),
        pltpu.SemaphoreType.DMA((2,)),
    ],
)
def _edge_kernel(u_hbm, v_hbm, u2_hbm, v2_hbm, su_hbm, sv_hbm,
                 src_hbm, dst_hbm, out_hbm,
                 idx_s, idx_d, su_t, sv_t, u_buf, v_buf, part, ob,
                 sem_u, sem_v):
    cid = lax.axis_index("c")
    sid = lax.axis_index("s")
    nb = jnp.where(cid == 0, NB0, NB1)
    ebase = jnp.where(cid == 0, sid * ET0, NS * ET0 + sid * ET1)

    pltpu.sync_copy(src_hbm.at[pl.ds(ebase, ET0)], idx_s)
    pltpu.sync_copy(dst_hbm.at[pl.ds(ebase, ET0)], idx_d)
    pltpu.sync_copy(su_hbm, su_t)
    pltpu.sync_copy(sv_hbm, sv_t)

    def _issue(j, s):
        @pl.when(cid == 0)
        def _from_copy0():
            pltpu.async_copy(u_hbm.at[idx_s.at[pl.ds(j * BB, BB)]],
                             u_buf.at[s], sem_u.at[s])
            pltpu.async_copy(v_hbm.at[idx_d.at[pl.ds(j * BB, BB)]],
                             v_buf.at[s], sem_v.at[s])

        @pl.when(cid == 1)
        def _from_copy1():
            pltpu.async_copy(u2_hbm.at[idx_s.at[pl.ds(j * BB, BB)]],
                             u_buf.at[s], sem_u.at[s])
            pltpu.async_copy(v2_hbm.at[idx_d.at[pl.ds(j * BB, BB)]],
                             v_buf.at[s], sem_v.at[s])

    def _wait(s):
        pltpu.make_async_copy(u_hbm.at[idx_s.at[pl.ds(0, BB)]],
                              u_buf.at[s], sem_u.at[s]).wait()
        pltpu.make_async_copy(v_hbm.at[idx_d.at[pl.ds(0, BB)]],
                              v_buf.at[s], sem_v.at[s]).wait()

    _issue(0, 0)

    lanes = lax.iota(jnp.int32, L)

    def _block(jj, carry):
        for s in range(2):
            j = jj * 2 + s

            @pl.when(j < nb - 1)
            def _next():
                _issue(j + 1, 1 - s)

            _wait(s)

            @plsc.parallel_loop(0, BB, 1, unroll=4)
            def _edge(e):
                acc = jnp.zeros((L,), jnp.float32)
                for cch in range(C // L):
                    uw = u_buf[s, e, pl.ds(cch * L, L)]
                    vw = v_buf[s, e, pl.ds(cch * L, L)]
                    ua, ub = plsc.unpack(plsc.bitcast(uw, jnp.bfloat16),
                                         format=plsc.PackFormat.INTERLEAVED)
                    va, vb = plsc.unpack(plsc.bitcast(vw, jnp.bfloat16),
                                         format=plsc.PackFormat.INTERLEAVED)
                    acc = acc + ua * va + ub * vb
                part[pl.ds(e * L, L)] = acc

            for g in range(BB // L):
                rows = (g * L + lanes) * L
                dots = plsc.load_gather(part, [rows])
                for cch in range(1, L):
                    dots = dots + plsc.load_gather(part, [rows + cch])
                sg = plsc.load_gather(su_t, [idx_s[pl.ds(j * BB + g * L, L)]])
                dg = plsc.load_gather(sv_t, [idx_d[pl.ds(j * BB + g * L, L)]])
                val = dots + sg + dg
                ob[s, pl.ds(g * L, L)] = 1.0 / (1.0 + jnp.exp(-val))

            pltpu.sync_copy(ob.at[s], out_hbm.at[pl.ds(ebase + j * BB, BB)])
        return carry
    lax.fori_loop(0, nb // 2, _block, 0)


def kernel(z_in, z_out, z_self, edge_index, W_in, b_in, W_out, b_out):
    ei = edge_index.astype(jnp.int32)
    src, dst = ei[0], ei[1]

    src2d = src.reshape(E // KC, KC)
    dst2d = dst.reshape(E // KC, KC)
    zeros_np = jnp.zeros((NP,), jnp.float32)
    cnts = _bincount_kernel(src2d, dst2d, zeros_np)

    zol = jnp.pad(z_out[:, C], (0, NP - N)).reshape(NP // 128, 128)
    zil = jnp.pad(z_in[:, C], (0, NP - N)).reshape(NP // 128, 128)
    su, sv = _scalars_call(cnts.reshape(NC, 2, NP // 128, 128), zol, zil)

    U, V, U2, V2 = _tables_call(z_out[:, :C], z_in[:, :C], z_self,
                                W_in, W_out,
                                b_in.reshape(1, C), b_out.reshape(1, C))

    pad = jnp.zeros((EPAD - E,), jnp.int32)
    srcp = jnp.concatenate([src, pad])
    dstp = jnp.concatenate([dst, pad])
    outp = _edge_kernel(U, V, U2, V2, su.reshape(-1), sv.reshape(-1),
                        srcp, dstp)
    return outp[:E]


# speedup vs baseline: 1.0571x; 1.0571x over previous
import functools

import jax
import jax.numpy as jnp
from jax import lax
from jax.experimental import pallas as pl
from jax.experimental.pallas import tpu as pltpu, tpu_sc as plsc

N = 10000
E = 320000
C = 128
D = 2 * C
ALPHA = 0.5
BETA = 0.5

NC, NS, L = 2, 16, 16
NW = NC * NS

NP = 10240
EP = 327680
BB = 128
NB0 = 80
NB1 = 80
ET0 = NB0 * BB
ET1 = NB1 * BB
EPAD = EP + (ET0 - ET1)
KC = 125
KR = E // NW // KC

_mesh = plsc.VectorSubcoreMesh(
    core_axis_name="c", subcore_axis_name="s", num_cores=NC, num_subcores=NS)


@functools.partial(
    pl.kernel,
    out_type=jax.ShapeDtypeStruct((NC, 2, NP), jnp.float32),
    mesh=_mesh,
    compiler_params=pltpu.CompilerParams(needs_layout_passes=False),
    scratch_types=[
        pltpu.VMEM((KR, KC), jnp.int32),
        pltpu.VMEM((KR, KC), jnp.int32),
        pltpu.VMEM((128,), jnp.float32),
        pltpu.VMEM_SHARED((NP,), jnp.float32),
        pltpu.VMEM_SHARED((NP,), jnp.float32),
        pltpu.SemaphoreType.DMA,
        pltpu.SemaphoreType.DMA,
    ],
)
def _bincount_kernel(src_hbm, dst_hbm, zeros_hbm, cnt_hbm,
                     idx_s, idx_d, ones_v, cnt_s_sh, cnt_d_sh, sem_s, sem_d):
    cid = lax.axis_index("c")
    sid = lax.axis_index("s")
    wid = sid * NC + cid

    @pl.when(sid == 0)
    def _zero():
        pltpu.sync_copy(zeros_hbm, cnt_s_sh)
        pltpu.sync_copy(zeros_hbm, cnt_d_sh)

    for k in range(128 // L):
        ones_v[pl.ds(k * L, L)] = jnp.ones((L,), jnp.float32)

    pltpu.sync_copy(src_hbm.at[pl.ds(wid * KR, KR)], idx_s)
    pltpu.sync_copy(dst_hbm.at[pl.ds(wid * KR, KR)], idx_d)
    plsc.subcore_barrier()

    ones_c = ones_v.at[pl.ds(0, KC)]

    def _issue(j, carry):
        pltpu.async_copy(ones_c, cnt_s_sh.at[idx_s.at[j]], sem_s, add=True)
        pltpu.async_copy(ones_c, cnt_d_sh.at[idx_d.at[j]], sem_d, add=True)
        return carry
    lax.fori_loop(0, KR, _issue, 0)

    def _drain(j, carry):
        pltpu.make_async_copy(ones_c, cnt_s_sh.at[idx_s.at[0]], sem_s).wait()
        pltpu.make_async_copy(ones_c, cnt_d_sh.at[idx_d.at[0]], sem_d).wait()
        return carry
    lax.fori_loop(0, KR, _drain, 0)

    plsc.subcore_barrier()

    @pl.when(sid == 0)
    def _writeback():
        pltpu.sync_copy(cnt_s_sh, cnt_hbm.at[cid, 0])
        pltpu.sync_copy(cnt_d_sh, cnt_hbm.at[cid, 1])


def _scalars_body(cnts_ref, zol_ref, zil_ref, su_ref, sv_ref):
    cs = cnts_ref[0, 0] + cnts_ref[1, 0]
    cd = cnts_ref[0, 1] + cnts_ref[1, 1]
    occ = cs + cd
    denom = jnp.sum((occ > 0).astype(jnp.float32)) - 1.0
    dego = cs / denom
    dego = dego / jnp.maximum(jnp.sqrt(jnp.sum(dego * dego)), 1e-12)
    degi = cd / denom
    degi = degi / jnp.maximum(jnp.sqrt(jnp.sum(degi * degi)), 1e-12)
    su_ref[...] = BETA * (dego + zol_ref[...])
    sv_ref[...] = BETA * (degi + zil_ref[...])


def _scalars_call(cnts, zol, zil):
    return pl.pallas_call(
        _scalars_body,
        out_shape=(jax.ShapeDtypeStruct((NP // 128, 128), jnp.float32),
                   jax.ShapeDtypeStruct((NP // 128, 128), jnp.float32)),
    )(cnts, zol, zil)


_RB = 1000


def _pack_words(lo_f32, hi_f32):
    lo = lax.bitcast_convert_type(lo_f32.astype(jnp.bfloat16),
                                  jnp.uint16).astype(jnp.uint32)
    hi = lax.bitcast_convert_type(hi_f32.astype(jnp.bfloat16),
                                  jnp.uint16).astype(jnp.uint32)
    return lo | (hi << 16)


def _tables_body(zo_ref, zi_ref, zs_ref, wi_ref, wo_ref, bi_ref, bo_ref,
                 u_ref, v_ref, u2_ref, v2_ref):
    zs = zs_ref[...]
    dn = (((1,), (1,)), ((), ()))
    a_in = lax.dot_general(zs, wi_ref[...], dn,
                           preferred_element_type=jnp.float32) + bi_ref[...]
    a_out = lax.dot_general(zs, wo_ref[...], dn,
                            preferred_element_type=jnp.float32) + bo_ref[...]
    uw = _pack_words(ALPHA * zo_ref[...], ALPHA * a_in)
    vw = _pack_words(a_out, zi_ref[...])
    u_ref[...] = uw
    v_ref[...] = vw
    u2_ref[...] = uw
    v2_ref[...] = vw


def _tables_call(zo, zi, zs, wi, wo, bi, bo):
    nsteps = N // _RB
    row_spec = pl.BlockSpec((_RB, C), lambda i: (i, 0))
    fix2 = pl.BlockSpec((C, C), lambda i: (0, 0))
    fixb = pl.BlockSpec((1, C), lambda i: (0, 0))
    return pl.pallas_call(
        _tables_body,
        grid=(nsteps,),
        in_specs=[row_spec, row_spec, row_spec, fix2, fix2, fixb, fixb],
        out_specs=(pl.BlockSpec((_RB, C), lambda i: (i, 0)),
                   pl.BlockSpec((_RB, C), lambda i: (i, 0)),
                   pl.BlockSpec((_RB, C), lambda i: (i, 0)),
                   pl.BlockSpec((_RB, C), lambda i: (i, 0))),
        out_shape=(jax.ShapeDtypeStruct((N, C), jnp.uint32),
                   jax.ShapeDtypeStruct((N, C), jnp.uint32),
                   jax.ShapeDtypeStruct((N, C), jnp.uint32),
                   jax.ShapeDtypeStruct((N, C), jnp.uint32)),
    )(zo, zi, zs, wi, wo, bi, bo)


@functools.partial(
    pl.kernel,
    out_type=jax.ShapeDtypeStruct((EP,), jnp.float32),
    mesh=_mesh,
    compiler_params=pltpu.CompilerParams(needs_layout_passes=False),
    scratch_types=[
        pltpu.VMEM((ET0,), jnp.int32),
        pltpu.VMEM((ET0,), jnp.int32),
        pltpu.VMEM((NP,), jnp.float32),
        pltpu.VMEM((NP,), jnp.float32),
        pltpu.VMEM((2, BB, C), jnp.uint32),
        pltpu.VMEM((2, BB, C), jnp.uint32),
        pltpu.VMEM((BB * L,), jnp.float32),
        pltpu.VMEM((2, BB), jnp.float32),
        pltpu.SemaphoreType.DMA((2,)),
        pltpu.SemaphoreType.DMA((2,)),
    ],
)
def _edge_kernel(u_hbm, v_hbm, u2_hbm, v2_hbm, su_hbm, sv_hbm,
                 src_hbm, dst_hbm, out_hbm,
                 idx_s, idx_d, su_t, sv_t, u_buf, v_buf, part, ob,
                 sem_u, sem_v):
    cid = lax.axis_index("c")
    sid = lax.axis_index("s")
    nb = jnp.where(cid == 0, NB0, NB1)
    ebase = jnp.where(cid == 0, sid * ET0, NS * ET0 + sid * ET1)

    pltpu.sync_copy(src_hbm.at[pl.ds(ebase, ET0)], idx_s)
    pltpu.sync_copy(dst_hbm.at[pl.ds(ebase, ET0)], idx_d)
    pltpu.sync_copy(su_hbm, su_t)
    pltpu.sync_copy(sv_hbm, sv_t)

    def _issue(j, s):
        @pl.when(cid == 0)
        def _from_copy0():
            pltpu.async_copy(u_hbm.at[idx_s.at[pl.ds(j * BB, BB)]],
                             u_buf.at[s], sem_u.at[s])
            pltpu.async_copy(v_hbm.at[idx_d.at[pl.ds(j * BB, BB)]],
                             v_buf.at[s], sem_v.at[s])

        @pl.when(cid == 1)
        def _from_copy1():
            pltpu.async_copy(u2_hbm.at[idx_s.at[pl.ds(j * BB, BB)]],
                             u_buf.at[s], sem_u.at[s])
            pltpu.async_copy(v2_hbm.at[idx_d.at[pl.ds(j * BB, BB)]],
                             v_buf.at[s], sem_v.at[s])

    def _wait(s):
        pltpu.make_async_copy(u_hbm.at[idx_s.at[pl.ds(0, BB)]],
                              u_buf.at[s], sem_u.at[s]).wait()
        pltpu.make_async_copy(v_hbm.at[idx_d.at[pl.ds(0, BB)]],
                              v_buf.at[s], sem_v.at[s]).wait()

    _issue(0, 0)

    lanes = lax.iota(jnp.int32, L)

    def _block(jj, carry):
        for s in range(2):
            j = jj * 2 + s

            @pl.when(j < nb - 1)
            def _next():
                _issue(j + 1, 1 - s)

            _wait(s)

            @plsc.parallel_loop(0, BB, 1, unroll=4)
            def _edge(e):
                acc = jnp.zeros((L,), jnp.float32)
                for cch in range(C // L):
                    uw = u_buf[s, e, pl.ds(cch * L, L)]
                    vw = v_buf[s, e, pl.ds(cch * L, L)]
                    ua, ub = plsc.unpack(plsc.bitcast(uw, jnp.bfloat16),
                                         format=plsc.PackFormat.INTERLEAVED)
                    va, vb = plsc.unpack(plsc.bitcast(vw, jnp.bfloat16),
                                         format=plsc.PackFormat.INTERLEAVED)
                    acc = acc + ua * va + ub * vb
                part[pl.ds(e * L, L)] = acc

            for g in range(BB // L):
                rows = (g * L + lanes) * L
                dots = plsc.load_gather(part, [rows])
                for cch in range(1, L):
                    dots = dots + plsc.load_gather(part, [rows + cch])
                sg = plsc.load_gather(su_t, [idx_s[pl.ds(j * BB + g * L, L)]])
                dg = plsc.load_gather(sv_t, [idx_d[pl.ds(j * BB + g * L, L)]])
                val = dots + sg + dg
                ob[s, pl.ds(g * L, L)] = 1.0 / (1.0 + jnp.exp(-val))

            pltpu.sync_copy(ob.at[s], out_hbm.at[pl.ds(ebase + j * BB, BB)])
        return carry
    lax.fori_loop(0, nb // 2, _block, 0)


def kernel(z_in, z_out, z_self, edge_index, W_in, b_in, W_out, b_out):
    ei = edge_index.astype(jnp.int32)
    src, dst = ei[0], ei[1]

    src2d = src.reshape(E // KC, KC)
    dst2d = dst.reshape(E // KC, KC)
    zeros_np = jnp.zeros((NP,), jnp.float32)
    cnts = _bincount_kernel(src2d, dst2d, zeros_np)

    zol = jnp.pad(z_out[:, C], (0, NP - N)).reshape(NP // 128, 128)
    zil = jnp.pad(z_in[:, C], (0, NP - N)).reshape(NP // 128, 128)
    su, sv = _scalars_call(cnts.reshape(NC, 2, NP // 128, 128), zol, zil)

    U, V, U2, V2 = _tables_call(z_out[:, :C], z_in[:, :C], z_self,
                                W_in, W_out,
                                b_in.reshape(1, C), b_out.reshape(1, C))

    pad = jnp.zeros((EPAD - E,), jnp.int32)
    srcp = jnp.concatenate([src, pad])
    dstp = jnp.concatenate([dst, pad])
    outp = _edge_kernel(U, V, U2, V2, su.reshape(-1), sv.reshape(-1),
                        srcp, dstp)
    return outp[:E]


# speedup vs baseline: 1.1922x; 1.1278x over previous
import functools

import jax
import jax.numpy as jnp
from jax import lax
from jax.experimental import pallas as pl
from jax.experimental.pallas import tpu as pltpu, tpu_sc as plsc

N = 10000
E = 320000
C = 128
D = 2 * C
ALPHA = 0.5
BETA = 0.5

NC, NS, L = 2, 16, 16
NW = NC * NS

NP = 10240
EP = 327680
BB = 128
NB0 = 132
NB1 = 28
ET0 = NB0 * BB
ET1 = NB1 * BB
EPAD = EP + (ET0 - ET1)
KC = 125
KR = E // NW // KC

_mesh = plsc.VectorSubcoreMesh(
    core_axis_name="c", subcore_axis_name="s", num_cores=NC, num_subcores=NS)


@functools.partial(
    pl.kernel,
    out_type=jax.ShapeDtypeStruct((NC, 2, NP), jnp.float32),
    mesh=_mesh,
    compiler_params=pltpu.CompilerParams(needs_layout_passes=False),
    scratch_types=[
        pltpu.VMEM((KR, KC), jnp.int32),
        pltpu.VMEM((KR, KC), jnp.int32),
        pltpu.VMEM((128,), jnp.float32),
        pltpu.VMEM_SHARED((NP,), jnp.float32),
        pltpu.VMEM_SHARED((NP,), jnp.float32),
        pltpu.SemaphoreType.DMA,
        pltpu.SemaphoreType.DMA,
    ],
)
def _bincount_kernel(src_hbm, dst_hbm, zeros_hbm, cnt_hbm,
                     idx_s, idx_d, ones_v, cnt_s_sh, cnt_d_sh, sem_s, sem_d):
    cid = lax.axis_index("c")
    sid = lax.axis_index("s")
    wid = sid * NC + cid

    @pl.when(sid == 0)
    def _zero():
        pltpu.sync_copy(zeros_hbm, cnt_s_sh)
        pltpu.sync_copy(zeros_hbm, cnt_d_sh)

    for k in range(128 // L):
        ones_v[pl.ds(k * L, L)] = jnp.ones((L,), jnp.float32)

    pltpu.sync_copy(src_hbm.at[pl.ds(wid * KR, KR)], idx_s)
    pltpu.sync_copy(dst_hbm.at[pl.ds(wid * KR, KR)], idx_d)
    plsc.subcore_barrier()

    ones_c = ones_v.at[pl.ds(0, KC)]

    def _issue(j, carry):
        pltpu.async_copy(ones_c, cnt_s_sh.at[idx_s.at[j]], sem_s, add=True)
        pltpu.async_copy(ones_c, cnt_d_sh.at[idx_d.at[j]], sem_d, add=True)
        return carry
    lax.fori_loop(0, KR, _issue, 0)

    def _drain(j, carry):
        pltpu.make_async_copy(ones_c, cnt_s_sh.at[idx_s.at[0]], sem_s).wait()
        pltpu.make_async_copy(ones_c, cnt_d_sh.at[idx_d.at[0]], sem_d).wait()
        return carry
    lax.fori_loop(0, KR, _drain, 0)

    plsc.subcore_barrier()

    @pl.when(sid == 0)
    def _writeback():
        pltpu.sync_copy(cnt_s_sh, cnt_hbm.at[cid, 0])
        pltpu.sync_copy(cnt_d_sh, cnt_hbm.at[cid, 1])


def _scalars_body(cnts_ref, zol_ref, zil_ref, su_ref, sv_ref):
    cs = cnts_ref[0, 0] + cnts_ref[1, 0]
    cd = cnts_ref[0, 1] + cnts_ref[1, 1]
    occ = cs + cd
    denom = jnp.sum((occ > 0).astype(jnp.float32)) - 1.0
    dego = cs / denom
    dego = dego / jnp.maximum(jnp.sqrt(jnp.sum(dego * dego)), 1e-12)
    degi = cd / denom
    degi = degi / jnp.maximum(jnp.sqrt(jnp.sum(degi * degi)), 1e-12)
    su_ref[...] = BETA * (dego + zol_ref[...])
    sv_ref[...] = BETA * (degi + zil_ref[...])


def _scalars_call(cnts, zol, zil):
    return pl.pallas_call(
        _scalars_body,
        out_shape=(jax.ShapeDtypeStruct((NP // 128, 128), jnp.float32),
                   jax.ShapeDtypeStruct((NP // 128, 128), jnp.float32)),
    )(cnts, zol, zil)


_RB = 1000


def _pack_words(lo_f32, hi_f32):
    lo = lax.bitcast_convert_type(lo_f32.astype(jnp.bfloat16),
                                  jnp.uint16).astype(jnp.uint32)
    hi = lax.bitcast_convert_type(hi_f32.astype(jnp.bfloat16),
                                  jnp.uint16).astype(jnp.uint32)
    return lo | (hi << 16)


def _tables_body(zo_ref, zi_ref, zs_ref, wi_ref, wo_ref, bi_ref, bo_ref,
                 u_ref, v_ref, u2_ref, v2_ref):
    zs = zs_ref[...]
    dn = (((1,), (1,)), ((), ()))
    a_in = lax.dot_general(zs, wi_ref[...], dn,
                           preferred_element_type=jnp.float32) + bi_ref[...]
    a_out = lax.dot_general(zs, wo_ref[...], dn,
                            preferred_element_type=jnp.float32) + bo_ref[...]
    uw = _pack_words(ALPHA * zo_ref[...], ALPHA * a_in)
    vw = _pack_words(a_out, zi_ref[...])
    u_ref[...] = uw
    v_ref[...] = vw
    u2_ref[...] = uw
    v2_ref[...] = vw


def _tables_call(zo, zi, zs, wi, wo, bi, bo):
    nsteps = N // _RB
    row_spec = pl.BlockSpec((_RB, C), lambda i: (i, 0))
    fix2 = pl.BlockSpec((C, C), lambda i: (0, 0))
    fixb = pl.BlockSpec((1, C), lambda i: (0, 0))
    return pl.pallas_call(
        _tables_body,
        grid=(nsteps,),
        in_specs=[row_spec, row_spec, row_spec, fix2, fix2, fixb, fixb],
        out_specs=(pl.BlockSpec((_RB, C), lambda i: (i, 0)),
                   pl.BlockSpec((_RB, C), lambda i: (i, 0)),
                   pl.BlockSpec((_RB, C), lambda i: (i, 0)),
                   pl.BlockSpec((_RB, C), lambda i: (i, 0))),
        out_shape=(jax.ShapeDtypeStruct((N, C), jnp.uint32),
                   jax.ShapeDtypeStruct((N, C), jnp.uint32),
                   jax.ShapeDtypeStruct((N, C), jnp.uint32),
                   jax.ShapeDtypeStruct((N, C), jnp.uint32)),
    )(zo, zi, zs, wi, wo, bi, bo)


@functools.partial(
    pl.kernel,
    out_type=jax.ShapeDtypeStruct((EP,), jnp.float32),
    mesh=_mesh,
    compiler_params=pltpu.CompilerParams(needs_layout_passes=False),
    scratch_types=[
        pltpu.VMEM((ET0,), jnp.int32),
        pltpu.VMEM((ET0,), jnp.int32),
        pltpu.VMEM((NP,), jnp.float32),
        pltpu.VMEM((NP,), jnp.float32),
        pltpu.VMEM((2, BB, C), jnp.uint32),
        pltpu.VMEM((2, BB, C), jnp.uint32),
        pltpu.VMEM((BB * L,), jnp.float32),
        pltpu.VMEM((2, BB), jnp.float32),
        pltpu.SemaphoreType.DMA((2,)),
        pltpu.SemaphoreType.DMA((2,)),
    ],
)
def _edge_kernel(u_hbm, v_hbm, u2_hbm, v2_hbm, su_hbm, sv_hbm,
                 src_hbm, dst_hbm, out_hbm,
                 idx_s, idx_d, su_t, sv_t, u_buf, v_buf, part, ob,
                 sem_u, sem_v):
    cid = lax.axis_index("c")
    sid = lax.axis_index("s")
    nb = jnp.where(cid == 0, NB0, NB1)
    ebase = jnp.where(cid == 0, sid * ET0, NS * ET0 + sid * ET1)

    pltpu.sync_copy(src_hbm.at[pl.ds(ebase, ET0)], idx_s)
    pltpu.sync_copy(dst_hbm.at[pl.ds(ebase, ET0)], idx_d)
    pltpu.sync_copy(su_hbm, su_t)
    pltpu.sync_copy(sv_hbm, sv_t)

    def _issue(j, s):
        @pl.when(cid == 0)
        def _from_copy0():
            pltpu.async_copy(u_hbm.at[idx_s.at[pl.ds(j * BB, BB)]],
                             u_buf.at[s], sem_u.at[s])
            pltpu.async_copy(v_hbm.at[idx_d.at[pl.ds(j * BB, BB)]],
                             v_buf.at[s], sem_v.at[s])

        @pl.when(cid == 1)
        def _from_copy1():
            pltpu.async_copy(u2_hbm.at[idx_s.at[pl.ds(j * BB, BB)]],
                             u_buf.at[s], sem_u.at[s])
            pltpu.async_copy(v2_hbm.at[idx_d.at[pl.ds(j * BB, BB)]],
                             v_buf.at[s], sem_v.at[s])

    def _wait(s):
        pltpu.make_async_copy(u_hbm.at[idx_s.at[pl.ds(0, BB)]],
                              u_buf.at[s], sem_u.at[s]).wait()
        pltpu.make_async_copy(v_hbm.at[idx_d.at[pl.ds(0, BB)]],
                              v_buf.at[s], sem_v.at[s]).wait()

    _issue(0, 0)

    lanes = lax.iota(jnp.int32, L)

    def _block(jj, carry):
        for s in range(2):
            j = jj * 2 + s

            @pl.when(j < nb - 1)
            def _next():
                _issue(j + 1, 1 - s)

            _wait(s)

            @plsc.parallel_loop(0, BB, 1, unroll=4)
            def _edge(e):
                acc = jnp.zeros((L,), jnp.float32)
                for cch in range(C // L):
                    uw = u_buf[s, e, pl.ds(cch * L, L)]
                    vw = v_buf[s, e, pl.ds(cch * L, L)]
                    ua, ub = plsc.unpack(plsc.bitcast(uw, jnp.bfloat16),
                                         format=plsc.PackFormat.INTERLEAVED)
                    va, vb = plsc.unpack(plsc.bitcast(vw, jnp.bfloat16),
                                         format=plsc.PackFormat.INTERLEAVED)
                    acc = acc + ua * va + ub * vb
                part[pl.ds(e * L, L)] = acc

            for g in range(BB // L):
                rows = (g * L + lanes) * L
                dots = plsc.load_gather(part, [rows])
                for cch in range(1, L):
                    dots = dots + plsc.load_gather(part, [rows + cch])
                sg = plsc.load_gather(su_t, [idx_s[pl.ds(j * BB + g * L, L)]])
                dg = plsc.load_gather(sv_t, [idx_d[pl.ds(j * BB + g * L, L)]])
                val = dots + sg + dg
                ob[s, pl.ds(g * L, L)] = 1.0 / (1.0 + jnp.exp(-val))

            pltpu.sync_copy(ob.at[s], out_hbm.at[pl.ds(ebase + j * BB, BB)])
        return carry
    lax.fori_loop(0, nb // 2, _block, 0)


def kernel(z_in, z_out, z_self, edge_index, W_in, b_in, W_out, b_out):
    ei = edge_index.astype(jnp.int32)
    src, dst = ei[0], ei[1]

    src2d = src.reshape(E // KC, KC)
    dst2d = dst.reshape(E // KC, KC)
    zeros_np = jnp.zeros((NP,), jnp.float32)
    cnts = _bincount_kernel(src2d, dst2d, zeros_np)

    zol = jnp.pad(z_out[:, C], (0, NP - N)).reshape(NP // 128, 128)
    zil = jnp.pad(z_in[:, C], (0, NP - N)).reshape(NP // 128, 128)
    su, sv = _scalars_call(cnts.reshape(NC, 2, NP // 128, 128), zol, zil)

    U, V, U2, V2 = _tables_call(z_out[:, :C], z_in[:, :C], z_self,
                                W_in, W_out,
                                b_in.reshape(1, C), b_out.reshape(1, C))

    pad = jnp.zeros((EPAD - E,), jnp.int32)
    srcp = jnp.concatenate([src, pad])
    dstp = jnp.concatenate([dst, pad])
    outp = _edge_kernel(U, V, U2, V2, su.reshape(-1), sv.reshape(-1),
                        srcp, dstp)
    return outp[:E]


# speedup vs baseline: 1.2057x; 1.0113x over previous
import functools

import jax
import jax.numpy as jnp
from jax import lax
from jax.experimental import pallas as pl
from jax.experimental.pallas import tpu as pltpu, tpu_sc as plsc

N = 10000
E = 320000
C = 128
D = 2 * C
ALPHA = 0.5
BETA = 0.5

NC, NS, L = 2, 16, 16
NW = NC * NS

NP = 10240
EP = 327680
BB = 128
NB0 = 140
NB1 = 20
ET0 = NB0 * BB
ET1 = NB1 * BB
EPAD = EP + (ET0 - ET1)
KC = 125
KR = E // NW // KC

_mesh = plsc.VectorSubcoreMesh(
    core_axis_name="c", subcore_axis_name="s", num_cores=NC, num_subcores=NS)


@functools.partial(
    pl.kernel,
    out_type=jax.ShapeDtypeStruct((NC, 2, NP), jnp.float32),
    mesh=_mesh,
    compiler_params=pltpu.CompilerParams(needs_layout_passes=False),
    scratch_types=[
        pltpu.VMEM((KR, KC), jnp.int32),
        pltpu.VMEM((KR, KC), jnp.int32),
        pltpu.VMEM((128,), jnp.float32),
        pltpu.VMEM_SHARED((NP,), jnp.float32),
        pltpu.VMEM_SHARED((NP,), jnp.float32),
        pltpu.SemaphoreType.DMA,
        pltpu.SemaphoreType.DMA,
    ],
)
def _bincount_kernel(src_hbm, dst_hbm, zeros_hbm, cnt_hbm,
                     idx_s, idx_d, ones_v, cnt_s_sh, cnt_d_sh, sem_s, sem_d):
    cid = lax.axis_index("c")
    sid = lax.axis_index("s")
    wid = sid * NC + cid

    @pl.when(sid == 0)
    def _zero():
        pltpu.sync_copy(zeros_hbm, cnt_s_sh)
        pltpu.sync_copy(zeros_hbm, cnt_d_sh)

    for k in range(128 // L):
        ones_v[pl.ds(k * L, L)] = jnp.ones((L,), jnp.float32)

    pltpu.sync_copy(src_hbm.at[pl.ds(wid * KR, KR)], idx_s)
    pltpu.sync_copy(dst_hbm.at[pl.ds(wid * KR, KR)], idx_d)
    plsc.subcore_barrier()

    ones_c = ones_v.at[pl.ds(0, KC)]

    def _issue(j, carry):
        pltpu.async_copy(ones_c, cnt_s_sh.at[idx_s.at[j]], sem_s, add=True)
        pltpu.async_copy(ones_c, cnt_d_sh.at[idx_d.at[j]], sem_d, add=True)
        return carry
    lax.fori_loop(0, KR, _issue, 0)

    def _drain(j, carry):
        pltpu.make_async_copy(ones_c, cnt_s_sh.at[idx_s.at[0]], sem_s).wait()
        pltpu.make_async_copy(ones_c, cnt_d_sh.at[idx_d.at[0]], sem_d).wait()
        return carry
    lax.fori_loop(0, KR, _drain, 0)

    plsc.subcore_barrier()

    @pl.when(sid == 0)
    def _writeback():
        pltpu.sync_copy(cnt_s_sh, cnt_hbm.at[cid, 0])
        pltpu.sync_copy(cnt_d_sh, cnt_hbm.at[cid, 1])


def _scalars_body(cnts_ref, zol_ref, zil_ref, su_ref, sv_ref):
    cs = cnts_ref[0, 0] + cnts_ref[1, 0]
    cd = cnts_ref[0, 1] + cnts_ref[1, 1]
    occ = cs + cd
    denom = jnp.sum((occ > 0).astype(jnp.float32)) - 1.0
    dego = cs / denom
    dego = dego / jnp.maximum(jnp.sqrt(jnp.sum(dego * dego)), 1e-12)
    degi = cd / denom
    degi = degi / jnp.maximum(jnp.sqrt(jnp.sum(degi * degi)), 1e-12)
    su_ref[...] = BETA * (dego + zol_ref[...])
    sv_ref[...] = BETA * (degi + zil_ref[...])


def _scalars_call(cnts, zol, zil):
    return pl.pallas_call(
        _scalars_body,
        out_shape=(jax.ShapeDtypeStruct((NP // 128, 128), jnp.float32),
                   jax.ShapeDtypeStruct((NP // 128, 128), jnp.float32)),
    )(cnts, zol, zil)


_RB = 1000


def _pack_words(lo_f32, hi_f32):
    lo = lax.bitcast_convert_type(lo_f32.astype(jnp.bfloat16),
                                  jnp.uint16).astype(jnp.uint32)
    hi = lax.bitcast_convert_type(hi_f32.astype(jnp.bfloat16),
                                  jnp.uint16).astype(jnp.uint32)
    return lo | (hi << 16)


def _tables_body(zo_ref, zi_ref, zs_ref, wi_ref, wo_ref, bi_ref, bo_ref,
                 u_ref, v_ref, u2_ref, v2_ref):
    zs = zs_ref[...]
    dn = (((1,), (1,)), ((), ()))
    a_in = lax.dot_general(zs, wi_ref[...], dn,
                           preferred_element_type=jnp.float32) + bi_ref[...]
    a_out = lax.dot_general(zs, wo_ref[...], dn,
                            preferred_element_type=jnp.float32) + bo_ref[...]
    uw = _pack_words(ALPHA * zo_ref[...], ALPHA * a_in)
    vw = _pack_words(a_out, zi_ref[...])
    u_ref[...] = uw
    v_ref[...] = vw
    u2_ref[...] = uw
    v2_ref[...] = vw


def _tables_call(zo, zi, zs, wi, wo, bi, bo):
    nsteps = N // _RB
    row_spec = pl.BlockSpec((_RB, C), lambda i: (i, 0))
    fix2 = pl.BlockSpec((C, C), lambda i: (0, 0))
    fixb = pl.BlockSpec((1, C), lambda i: (0, 0))
    return pl.pallas_call(
        _tables_body,
        grid=(nsteps,),
        in_specs=[row_spec, row_spec, row_spec, fix2, fix2, fixb, fixb],
        out_specs=(pl.BlockSpec((_RB, C), lambda i: (i, 0)),
                   pl.BlockSpec((_RB, C), lambda i: (i, 0)),
                   pl.BlockSpec((_RB, C), lambda i: (i, 0)),
                   pl.BlockSpec((_RB, C), lambda i: (i, 0))),
        out_shape=(jax.ShapeDtypeStruct((N, C), jnp.uint32),
                   jax.ShapeDtypeStruct((N, C), jnp.uint32),
                   jax.ShapeDtypeStruct((N, C), jnp.uint32),
                   jax.ShapeDtypeStruct((N, C), jnp.uint32)),
    )(zo, zi, zs, wi, wo, bi, bo)


@functools.partial(
    pl.kernel,
    out_type=jax.ShapeDtypeStruct((EP,), jnp.float32),
    mesh=_mesh,
    compiler_params=pltpu.CompilerParams(needs_layout_passes=False),
    scratch_types=[
        pltpu.VMEM((ET0,), jnp.int32),
        pltpu.VMEM((ET0,), jnp.int32),
        pltpu.VMEM((NP,), jnp.float32),
        pltpu.VMEM((NP,), jnp.float32),
        pltpu.VMEM((2, BB, C), jnp.uint32),
        pltpu.VMEM((2, BB, C), jnp.uint32),
        pltpu.VMEM((BB * L,), jnp.float32),
        pltpu.VMEM((2, BB), jnp.float32),
        pltpu.SemaphoreType.DMA((2,)),
        pltpu.SemaphoreType.DMA((2,)),
    ],
)
def _edge_kernel(u_hbm, v_hbm, u2_hbm, v2_hbm, su_hbm, sv_hbm,
                 src_hbm, dst_hbm, out_hbm,
                 idx_s, idx_d, su_t, sv_t, u_buf, v_buf, part, ob,
                 sem_u, sem_v):
    cid = lax.axis_index("c")
    sid = lax.axis_index("s")
    nb = jnp.where(cid == 0, NB0, NB1)
    ebase = jnp.where(cid == 0, sid * ET0, NS * ET0 + sid * ET1)

    pltpu.sync_copy(src_hbm.at[pl.ds(ebase, ET0)], idx_s)
    pltpu.sync_copy(dst_hbm.at[pl.ds(ebase, ET0)], idx_d)
    pltpu.sync_copy(su_hbm, su_t)
    pltpu.sync_copy(sv_hbm, sv_t)

    def _issue(j, s):
        @pl.when(cid == 0)
        def _from_copy0():
            pltpu.async_copy(u_hbm.at[idx_s.at[pl.ds(j * BB, BB)]],
                             u_buf.at[s], sem_u.at[s])
            pltpu.async_copy(v_hbm.at[idx_d.at[pl.ds(j * BB, BB)]],
                             v_buf.at[s], sem_v.at[s])

        @pl.when(cid == 1)
        def _from_copy1():
            pltpu.async_copy(u2_hbm.at[idx_s.at[pl.ds(j * BB, BB)]],
                             u_buf.at[s], sem_u.at[s])
            pltpu.async_copy(v2_hbm.at[idx_d.at[pl.ds(j * BB, BB)]],
                             v_buf.at[s], sem_v.at[s])

    def _wait(s):
        pltpu.make_async_copy(u_hbm.at[idx_s.at[pl.ds(0, BB)]],
                              u_buf.at[s], sem_u.at[s]).wait()
        pltpu.make_async_copy(v_hbm.at[idx_d.at[pl.ds(0, BB)]],
                              v_buf.at[s], sem_v.at[s]).wait()

    _issue(0, 0)

    lanes = lax.iota(jnp.int32, L)

    def _block(jj, carry):
        for s in range(2):
            j = jj * 2 + s

            @pl.when(j < nb - 1)
            def _next():
                _issue(j + 1, 1 - s)

            _wait(s)

            @plsc.parallel_loop(0, BB, 1, unroll=4)
            def _edge(e):
                acc = jnp.zeros((L,), jnp.float32)
                for cch in range(C // L):
                    uw = u_buf[s, e, pl.ds(cch * L, L)]
                    vw = v_buf[s, e, pl.ds(cch * L, L)]
                    ua, ub = plsc.unpack(plsc.bitcast(uw, jnp.bfloat16),
                                         format=plsc.PackFormat.INTERLEAVED)
                    va, vb = plsc.unpack(plsc.bitcast(vw, jnp.bfloat16),
                                         format=plsc.PackFormat.INTERLEAVED)
                    acc = acc + ua * va + ub * vb
                part[pl.ds(e * L, L)] = acc

            for g in range(BB // L):
                rows = (g * L + lanes) * L
                dots = plsc.load_gather(part, [rows])
                for cch in range(1, L):
                    dots = dots + plsc.load_gather(part, [rows + cch])
                sg = plsc.load_gather(su_t, [idx_s[pl.ds(j * BB + g * L, L)]])
                dg = plsc.load_gather(sv_t, [idx_d[pl.ds(j * BB + g * L, L)]])
                val = dots + sg + dg
                ob[s, pl.ds(g * L, L)] = 1.0 / (1.0 + jnp.exp(-val))

            pltpu.sync_copy(ob.at[s], out_hbm.at[pl.ds(ebase + j * BB, BB)])
        return carry
    lax.fori_loop(0, nb // 2, _block, 0)


def kernel(z_in, z_out, z_self, edge_index, W_in, b_in, W_out, b_out):
    ei = edge_index.astype(jnp.int32)
    src, dst = ei[0], ei[1]

    src2d = src.reshape(E // KC, KC)
    dst2d = dst.reshape(E // KC, KC)
    zeros_np = jnp.zeros((NP,), jnp.float32)
    cnts = _bincount_kernel(src2d, dst2d, zeros_np)

    zol = jnp.pad(z_out[:, C], (0, NP - N)).reshape(NP // 128, 128)
    zil = jnp.pad(z_in[:, C], (0, NP - N)).reshape(NP // 128, 128)
    su, sv = _scalars_call(cnts.reshape(NC, 2, NP // 128, 128), zol, zil)

    U, V, U2, V2 = _tables_call(z_out[:, :C], z_in[:, :C], z_self,
                                W_in, W_out,
                                b_in.reshape(1, C), b_out.reshape(1, C))

    pad = jnp.zeros((EPAD - E,), jnp.int32)
    srcp = jnp.concatenate([src, pad])
    dstp = jnp.concatenate([dst, pad])
    outp = _edge_kernel(U, V, U2, V2, su.reshape(-1), sv.reshape(-1),
                        srcp, dstp)
    return outp[:E]
